# Initial kernel scaffold; baseline (speedup 1.0000x reference)
import jax
import jax.numpy as jnp
from jax.experimental import pallas as pl


def kernel(hidden_states, Wg, W1, b1, W2, b2):
    raise NotImplementedError("write your pallas kernel here")


# speedup vs baseline: 7.9688x; 7.9688x over previous
import functools

import jax
import jax.numpy as jnp
from jax.experimental import pallas as pl
from jax.experimental.pallas import tpu as pltpu


def _moe_body(x_ref, wgt_ref, w1_ref, b1_ref, w2_ref, b2_ref, out_ref,
              xb_ref, i1_ref, i2_ref, a1_ref, a2_ref, *, num_experts):
    e = pl.program_id(0)
    fb = pl.program_id(1)

    @pl.when((e == 0) & (fb == 0))
    def _router():
        x = x_ref[...]
        xb_ref[...] = x.astype(jnp.bfloat16)
        logits = jnp.dot(x, wgt_ref[...], preferred_element_type=jnp.float32)
        lane = jax.lax.broadcasted_iota(jnp.int32, logits.shape, 1)
        valid = lane < num_experts
        logits = jnp.where(valid, logits, jnp.float32(-1e30))
        mx = jnp.max(logits, axis=1, keepdims=True)
        ex = jnp.where(valid, jnp.exp(logits - mx), 0.0)
        probs = ex / jnp.sum(ex, axis=1, keepdims=True)
        big = jnp.int32(logits.shape[1])
        p1 = jnp.max(probs, axis=1, keepdims=True)
        i1 = jnp.min(jnp.where(probs == p1, lane, big), axis=1, keepdims=True)
        probs2 = jnp.where(lane == i1, jnp.float32(-1.0), probs)
        p2 = jnp.max(probs2, axis=1, keepdims=True)
        i2 = jnp.min(jnp.where(probs2 == p2, lane, big), axis=1, keepdims=True)
        s = p1 + p2
        i1_ref[...] = i1
        i2_ref[...] = i2
        a1_ref[...] = p1 / s
        a2_ref[...] = p2 / s
        out_ref[...] = jnp.zeros_like(out_ref)

    c = (jnp.where(i1_ref[...] == e, a1_ref[...], 0.0)
         + jnp.where(i2_ref[...] == e, a2_ref[...], 0.0))

    h = jnp.dot(xb_ref[...], w1_ref[0].astype(jnp.bfloat16),
                preferred_element_type=jnp.float32)
    h = h + b1_ref[0]
    h = 0.5 * h * (1.0 + jax.lax.erf(h * jnp.float32(0.7071067811865476)))
    y = jnp.dot(h.astype(jnp.bfloat16), w2_ref[0].astype(jnp.bfloat16),
                preferred_element_type=jnp.float32)
    first_fb = (fb == 0).astype(jnp.float32)
    y = y + first_fb * b2_ref[0]
    out_ref[...] += c * y


def kernel(hidden_states, Wg, W1, b1, W2, b2):
    B, S, D = hidden_states.shape
    E, _, F = W1.shape
    T = B * S
    x = hidden_states.reshape(T, D)

    FB = 768
    nfb = F // FB

    wgt = jnp.zeros((D, 128), jnp.float32).at[:, :E].set(Wg.T)

    body = functools.partial(_moe_body, num_experts=E)
    out = pl.pallas_call(
        body,
        grid=(E, nfb),
        in_specs=[
            pl.BlockSpec((T, D), lambda e, f: (0, 0)),
            pl.BlockSpec((D, 128), lambda e, f: (0, 0)),
            pl.BlockSpec((1, D, FB), lambda e, f: (e, 0, f)),
            pl.BlockSpec((1, 1, FB), lambda e, f: (e, 0, f)),
            pl.BlockSpec((1, FB, D), lambda e, f: (e, f, 0)),
            pl.BlockSpec((1, 1, D), lambda e, f: (e, 0, 0)),
        ],
        out_specs=pl.BlockSpec((T, D), lambda e, f: (0, 0)),
        out_shape=jax.ShapeDtypeStruct((T, D), jnp.float32),
        scratch_shapes=[
            pltpu.VMEM((T, D), jnp.bfloat16),
            pltpu.VMEM((T, 1), jnp.int32),
            pltpu.VMEM((T, 1), jnp.int32),
            pltpu.VMEM((T, 1), jnp.float32),
            pltpu.VMEM((T, 1), jnp.float32),
        ],
        compiler_params=pltpu.CompilerParams(
            dimension_semantics=("arbitrary", "arbitrary"),
        ),
    )(x, wgt, W1, b1.reshape(E, 1, F), W2, b2.reshape(E, 1, D))
    return out.reshape(B, S, D)
